# Initial kernel scaffold; baseline (speedup 1.0000x reference)
import jax
import jax.numpy as jnp
from jax.experimental import pallas as pl


def kernel(scalar_feat, coord_feat, vec_feat, edge_index, msg_Wh, msg_Wu, msg_Wf, msg_bf, msg_Wg, msg_bg, upd_Wh, upd_Wu, upd_Wf, upd_bf, upd_Wg, upd_bg, ln1_g, ln1_b, ln2_g, ln2_b):
    raise NotImplementedError("write your pallas kernel here")


# speedup vs baseline: 14.3721x; 14.3721x over previous
import functools

import jax
import jax.numpy as jnp
from jax import lax
from jax.experimental import pallas as pl
from jax.experimental.pallas import tpu as pltpu
from jax.experimental.pallas import tpu_sc as plsc

DS = 128
DV = 16
RD = 16
RBF_DMAX = 3.5
TW = 192
MW = 176
K = 128
NW = 32


def _node_precompute(scalar_feat, Wf_s, bf):
    n = scalar_feat.shape[0]
    blk = 1000

    def body(sf_ref, w_ref, b_ref, p_ref):
        p_ref[...] = jnp.dot(sf_ref[...], w_ref[...],
                             preferred_element_type=jnp.float32) + b_ref[...]

    return pl.pallas_call(
        body,
        grid=(n // blk,),
        in_specs=[
            pl.BlockSpec((blk, DS), lambda i: (i, 0)),
            pl.BlockSpec((DS, DS), lambda i: (0, 0)),
            pl.BlockSpec((1, DS), lambda i: (0, 0)),
        ],
        out_specs=pl.BlockSpec((blk, DS), lambda i: (i, 0)),
        out_shape=jax.ShapeDtypeStruct((n, DS), jnp.float32),
    )(scalar_feat, Wf_s, bf)


def _edge_gather(T, C16, src, dst):
    e = src.shape[0]
    nchunk = e // K
    nloop = (nchunk + NW - 1) // NW
    mesh = plsc.VectorSubcoreMesh(core_axis_name="c", subcore_axis_name="s")

    @functools.partial(
        pl.kernel,
        out_type=(jax.ShapeDtypeStruct((e, TW), jnp.float32),
                  jax.ShapeDtypeStruct((e, RD), jnp.float32)),
        mesh=mesh,
        scratch_types=[
            pltpu.VMEM((K,), jnp.int32),
            pltpu.VMEM((K,), jnp.int32),
            pltpu.VMEM((K, TW), jnp.float32),
            pltpu.VMEM((K, RD), jnp.float32),
            pltpu.SemaphoreType.DMA,
            pltpu.SemaphoreType.DMA,
        ],
        compiler_params=pltpu.CompilerParams(use_tc_tiling_on_sc=False),
    )
    def gk(t_hbm, c_hbm, src_hbm, dst_hbm, g_out, gd_out,
           idx_s, idx_d, rows, rowsd, sem_a, sem_b):
        wid = lax.axis_index("s") * 2 + lax.axis_index("c")

        def step(i, _):
            chunk = wid + NW * i

            @pl.when(chunk < nchunk)
            def _():
                base = chunk * K
                pltpu.sync_copy(src_hbm.at[pl.ds(base, K)], idx_s)
                pltpu.sync_copy(dst_hbm.at[pl.ds(base, K)], idx_d)
                cp_a = pltpu.async_copy(t_hbm.at[idx_s], rows, sem_a)
                cp_b = pltpu.async_copy(c_hbm.at[idx_d], rowsd, sem_b)
                cp_a.wait()
                cp_b.wait()
                pltpu.sync_copy(rows, g_out.at[pl.ds(base, K)])
                pltpu.sync_copy(rowsd, gd_out.at[pl.ds(base, K)])

            return 0

        lax.fori_loop(0, nloop, step, 0)

    return gk(T, C16, src, dst)


def _edge_compute(G, Gd, Wh, Wu, Wf_d, Wf_h, Wg, bg, dmu):
    e = G.shape[0]
    blk = 512

    def body(g_ref, gd_ref, wh_ref, wu_ref, wfd_ref, wfh_ref, wg_ref, bg_ref,
             dmu_ref, m_ref):
        g = g_ref[...]
        xdiff = g[:, 176:179] - gd_ref[:, 0:3]
        d2 = jnp.sum(xdiff * xdiff, axis=1, keepdims=True)
        dij = jnp.sqrt(jnp.maximum(d2, 1e-8)) + 1e-8
        xdn = xdiff / dij
        t = (dij - dmu_ref[...]) / (RBF_DMAX / RD)
        d = jnp.exp(-(t * t))
        wh = wh_ref[...]
        wu = wu_ref[...]
        vh = []
        vu = []
        for c in range(3):
            xc = jnp.concatenate(
                [g[:, DS + 16 * c:DS + 16 * (c + 1)], xdn[:, c:c + 1]], axis=1)
            h = jnp.dot(xc, wh, preferred_element_type=jnp.float32)
            vh.append(h)
            vu.append(jnp.dot(h, wu, preferred_element_type=jnp.float32))
        sh = jnp.sqrt(jnp.maximum(
            vh[0] * vh[0] + vh[1] * vh[1] + vh[2] * vh[2], 1e-8))
        pre = (g[:, 0:DS]
               + jnp.dot(d, wfd_ref[...], preferred_element_type=jnp.float32)
               + jnp.dot(sh, wfh_ref[...], preferred_element_type=jnp.float32))
        feats = pre * jax.nn.sigmoid(pre)
        gate = jax.nn.sigmoid(
            jnp.dot(feats, wg_ref[...], preferred_element_type=jnp.float32)
            + bg_ref[...])
        m_ref[:, 0:DS] = feats
        for c in range(3):
            m_ref[:, DS + 16 * c:DS + 16 * (c + 1)] = gate * vu[c]

    full = lambda shape: pl.BlockSpec(shape, lambda i: tuple(0 for _ in shape))
    return pl.pallas_call(
        body,
        grid=(e // blk,),
        in_specs=[
            pl.BlockSpec((blk, TW), lambda i: (i, 0)),
            pl.BlockSpec((blk, RD), lambda i: (i, 0)),
            full((DV + 1, DV + 1)),
            full((DV + 1, DV)),
            full((RD, DS)),
            full((DV + 1, DS)),
            full((DS, DV)),
            full((1, DV)),
            full((1, RD)),
        ],
        out_specs=pl.BlockSpec((blk, MW), lambda i: (i, 0)),
        out_shape=jax.ShapeDtypeStruct((e, MW), jnp.float32),
    )(G, Gd, Wh, Wu, Wf_d, Wf_h, Wg, bg, dmu)


def _scatter_add(M, dst, zeros_blk, n_nodes):
    KS = 64
    e = M.shape[0]
    nchunk = e // KS
    per_core = nchunk // 2
    nloop_e = (per_core + 15) // 16
    nzfull = n_nodes // K
    ztail = n_nodes - nzfull * K
    nloop_z = (nzfull + 1 + 15) // 16
    mesh = plsc.VectorSubcoreMesh(core_axis_name="c", subcore_axis_name="s")

    @functools.partial(
        pl.kernel,
        out_type=jax.ShapeDtypeStruct((2 * n_nodes, MW), jnp.float32),
        mesh=mesh,
        scratch_types=[
            pltpu.VMEM_SHARED((n_nodes, MW), jnp.float32),
            pltpu.VMEM((KS,), jnp.int32),
            pltpu.VMEM((KS, MW), jnp.float32),
        ],
        compiler_params=pltpu.CompilerParams(use_tc_tiling_on_sc=False),
    )
    def sk(m_hbm, dst_hbm, z_hbm, out_hbm, acc, idx, rows):
        cid = lax.axis_index("c")
        sid = lax.axis_index("s")

        def zstep(i, _):
            j = sid + 16 * i

            @pl.when(j < nzfull)
            def _():
                pltpu.sync_copy(z_hbm, acc.at[pl.ds(j * K, K)])

            @pl.when(j == nzfull)
            def _():
                pltpu.sync_copy(z_hbm.at[pl.ds(0, ztail)],
                                acc.at[pl.ds(nzfull * K, ztail)])

            return 0

        lax.fori_loop(0, nloop_z, zstep, 0)
        plsc.subcore_barrier()

        def estep(i, _):
            t = sid + 16 * i

            @pl.when(t < per_core)
            def _():
                base = (cid * per_core + t) * KS
                pltpu.sync_copy(dst_hbm.at[pl.ds(base, KS)], idx)
                pltpu.sync_copy(m_hbm.at[pl.ds(base, KS)], rows)
                pltpu.sync_copy(rows, acc.at[idx], add=True)

            return 0

        lax.fori_loop(0, nloop_e, estep, 0)
        plsc.subcore_barrier()

        def ostep(i, _):
            j = sid + 16 * i

            @pl.when(j < nzfull)
            def _():
                pltpu.sync_copy(acc.at[pl.ds(j * K, K)],
                                out_hbm.at[pl.ds(cid * n_nodes + j * K, K)])

            @pl.when(j == nzfull)
            def _():
                pltpu.sync_copy(
                    acc.at[pl.ds(nzfull * K, ztail)],
                    out_hbm.at[pl.ds(cid * n_nodes + nzfull * K, ztail)])

            return 0

        lax.fori_loop(0, nloop_z, ostep, 0)

    return sk(M, dst, zeros_blk)


def _node_update(scalar_feat, vt, p0, p1, uWh, uWu, uWf_s, uWf_h, ubf, uWg,
                 ubg, ln1_g, ln1_b, ln2_g, ln2_b):
    n = scalar_feat.shape[0]
    blk = 1000

    def body(sf_ref, vt_ref, p0_ref, p1_ref, wh_ref, wu_ref, wfs_ref, wfh_ref,
             bf_ref, wg_ref, bg_ref, g1_ref, b1_ref, g2_ref, b2_ref,
             so_ref, vo_ref):
        sf = sf_ref[...] + p0_ref[:, 0:DS] + p1_ref[:, 0:DS]
        vfc = [vt_ref[:, 16 * c:16 * (c + 1)]
               + p0_ref[:, DS + 16 * c:DS + 16 * (c + 1)]
               + p1_ref[:, DS + 16 * c:DS + 16 * (c + 1)] for c in range(3)]

        def ln(sf, vfc, g, b):
            mu = jnp.mean(sf, axis=1, keepdims=True)
            xc = sf - mu
            var = jnp.mean(xc * xc, axis=1, keepdims=True)
            nf = xc / jnp.sqrt(var + 1e-5) * g + b
            vn2 = vfc[0] * vfc[0] + vfc[1] * vfc[1] + vfc[2] * vfc[2]
            vnorm = jnp.sqrt(jnp.mean(vn2, axis=1, keepdims=True) + 1e-5) + 1e-5
            return nf, [v / vnorm for v in vfc]

        sf, vfc = ln(sf, vfc, g1_ref[...], b1_ref[...])
        vh = [jnp.dot(v, wh_ref[...], preferred_element_type=jnp.float32)
              for v in vfc]
        vu = [jnp.dot(h, wu_ref[...], preferred_element_type=jnp.float32)
              for h in vh]
        sh = jnp.sqrt(jnp.maximum(
            vh[0] * vh[0] + vh[1] * vh[1] + vh[2] * vh[2], 1e-8))
        pre = (jnp.dot(sf, wfs_ref[...], preferred_element_type=jnp.float32)
               + jnp.dot(sh, wfh_ref[...], preferred_element_type=jnp.float32)
               + bf_ref[...])
        feats = pre * jax.nn.sigmoid(pre)
        gate = jax.nn.sigmoid(
            jnp.dot(feats, wg_ref[...], preferred_element_type=jnp.float32)
            + bg_ref[...])
        sf2 = sf + feats
        vfc2 = [vfc[c] + gate * vu[c] for c in range(3)]
        so, voc = ln(sf2, vfc2, g2_ref[...], b2_ref[...])
        so_ref[...] = so
        for c in range(3):
            vo_ref[:, 16 * c:16 * (c + 1)] = voc[c]

    full = lambda shape: pl.BlockSpec(shape, lambda i: tuple(0 for _ in shape))
    return pl.pallas_call(
        body,
        grid=(n // blk,),
        in_specs=[
            pl.BlockSpec((blk, DS), lambda i: (i, 0)),
            pl.BlockSpec((blk, 3 * DV), lambda i: (i, 0)),
            pl.BlockSpec((blk, MW), lambda i: (i, 0)),
            pl.BlockSpec((blk, MW), lambda i: (i, 0)),
            full((DV, DV)),
            full((DV, DV)),
            full((DS, DS)),
            full((DV, DS)),
            full((1, DS)),
            full((DS, DV)),
            full((1, DV)),
            full((1, DS)),
            full((1, DS)),
            full((1, DS)),
            full((1, DS)),
        ],
        out_specs=(pl.BlockSpec((blk, DS), lambda i: (i, 0)),
                   pl.BlockSpec((blk, 3 * DV), lambda i: (i, 0))),
        out_shape=(jax.ShapeDtypeStruct((n, DS), jnp.float32),
                   jax.ShapeDtypeStruct((n, 3 * DV), jnp.float32)),
    )(scalar_feat, vt, p0, p1, uWh, uWu, uWf_s, uWf_h, ubf, uWg, ubg,
      ln1_g, ln1_b, ln2_g, ln2_b)


def kernel(scalar_feat, coord_feat, vec_feat, edge_index, msg_Wh, msg_Wu,
           msg_Wf, msg_bf, msg_Wg, msg_bg, upd_Wh, upd_Wu, upd_Wf, upd_bf,
           upd_Wg, upd_bg, ln1_g, ln1_b, ln2_g, ln2_b):
    n = scalar_feat.shape[0]
    src = edge_index[0]
    dst = edge_index[1]

    P = _node_precompute(scalar_feat, msg_Wf[:DS], msg_bf.reshape(1, DS))
    vt = vec_feat.transpose(0, 2, 1).reshape(n, 3 * DV)
    pad = jnp.zeros((n, 13), jnp.float32)
    T = jnp.concatenate([P, vt, coord_feat, pad], axis=1)
    C16 = jnp.concatenate([coord_feat, pad], axis=1)

    G, Gd = _edge_gather(T, C16, src, dst)

    dmu = jnp.linspace(0.0, RBF_DMAX, RD).reshape(1, RD)
    M = _edge_compute(G, Gd, msg_Wh, msg_Wu, msg_Wf[DS:DS + RD],
                      msg_Wf[DS + RD:], msg_Wg, msg_bg.reshape(1, DV), dmu)

    zeros_blk = jnp.zeros((K, MW), jnp.float32)
    part = _scatter_add(M, dst, zeros_blk, n)
    p0 = part[:n]
    p1 = part[n:]

    so, vo = _node_update(
        scalar_feat, vt, p0, p1, upd_Wh, upd_Wu, upd_Wf[:DS], upd_Wf[DS:],
        upd_bf.reshape(1, DS), upd_Wg, upd_bg.reshape(1, DV),
        ln1_g.reshape(1, DS), ln1_b.reshape(1, DS),
        ln2_g.reshape(1, DS), ln2_b.reshape(1, DS))
    vout = vo.reshape(n, 3, DV).transpose(0, 2, 1)
    return so, vout


# speedup vs baseline: 32.6692x; 2.2731x over previous
import functools

import jax
import jax.numpy as jnp
from jax import lax
from jax.experimental import pallas as pl
from jax.experimental.pallas import tpu as pltpu
from jax.experimental.pallas import tpu_sc as plsc

DS = 128
DV = 16
RD = 16
RBF_DMAX = 3.5
LW = 128
K = 128
NW = 32


def _node_precompute(scalar_feat, Wf_s, bf):
    n = scalar_feat.shape[0]
    blk = 1000

    def body(sf_ref, w_ref, b_ref, p_ref):
        p_ref[...] = jnp.dot(sf_ref[...], w_ref[...],
                             preferred_element_type=jnp.float32) + b_ref[...]

    return pl.pallas_call(
        body,
        grid=(n // blk,),
        in_specs=[
            pl.BlockSpec((blk, DS), lambda i: (i, 0)),
            pl.BlockSpec((DS, DS), lambda i: (0, 0)),
            pl.BlockSpec((1, DS), lambda i: (0, 0)),
        ],
        out_specs=pl.BlockSpec((blk, DS), lambda i: (i, 0)),
        out_shape=jax.ShapeDtypeStruct((n, DS), jnp.float32),
    )(scalar_feat, Wf_s, bf)


def _edge_gather(T1, T2, C, src, dst):
    e = src.shape[0]
    nchunk = e // K
    nloop = (nchunk + NW - 1) // NW
    mesh = plsc.VectorSubcoreMesh(core_axis_name="c", subcore_axis_name="s")

    @functools.partial(
        pl.kernel,
        out_type=(jax.ShapeDtypeStruct((e, LW), jnp.float32),
                  jax.ShapeDtypeStruct((e, LW), jnp.float32),
                  jax.ShapeDtypeStruct((e, LW), jnp.float32)),
        mesh=mesh,
        scratch_types=[
            pltpu.VMEM((K,), jnp.int32),
            pltpu.VMEM((K,), jnp.int32),
            pltpu.VMEM((K, LW), jnp.float32),
            pltpu.VMEM((K, LW), jnp.float32),
            pltpu.VMEM((K, LW), jnp.float32),
            pltpu.SemaphoreType.DMA,
            pltpu.SemaphoreType.DMA,
            pltpu.SemaphoreType.DMA,
        ],
    )
    def gk(t1_hbm, t2_hbm, c_hbm, src_hbm, dst_hbm, g1_out, g2_out, gd_out,
           idx_s, idx_d, r1, r2, rd, sem1, sem2, sem3):
        wid = lax.axis_index("s") * 2 + lax.axis_index("c")

        def step(i, _):
            chunk = wid + NW * i

            @pl.when(chunk < nchunk)
            def _():
                base = chunk * K
                pltpu.sync_copy(src_hbm.at[pl.ds(base, K)], idx_s)
                pltpu.sync_copy(dst_hbm.at[pl.ds(base, K)], idx_d)
                cp1 = pltpu.async_copy(t1_hbm.at[idx_s], r1, sem1)
                cp2 = pltpu.async_copy(t2_hbm.at[idx_s], r2, sem2)
                cp3 = pltpu.async_copy(c_hbm.at[idx_d], rd, sem3)
                cp1.wait()
                cp2.wait()
                cp3.wait()
                pltpu.sync_copy(r1, g1_out.at[pl.ds(base, K)])
                pltpu.sync_copy(r2, g2_out.at[pl.ds(base, K)])
                pltpu.sync_copy(rd, gd_out.at[pl.ds(base, K)])

            return 0

        lax.fori_loop(0, nloop, step, 0)

    return gk(T1, T2, C, src, dst)


def _edge_compute(G1, G2, Gd, Wh, Wh16, Wu, Wf_d, Wf_h, Wg, bg, dmu):
    e = G1.shape[0]
    blk = 1280

    def body(g1_ref, g2_ref, gd_ref, wh_ref, wh16_ref, wu_ref, wfd_ref,
             wfh_ref, wg_ref, bg_ref, dmu_ref, ms_ref, mv_ref):
        g2 = g2_ref[...]
        xdiff = g2[:, 48:51] - gd_ref[:, 0:3]
        d2 = jnp.sum(xdiff * xdiff, axis=1, keepdims=True)
        dij = jnp.sqrt(jnp.maximum(d2, 1e-8)) + 1e-8
        xdn = xdiff / dij
        t = (dij - dmu_ref[...]) / (RBF_DMAX / RD)
        d = jnp.exp(-(t * t))
        wh = wh_ref[...]
        wh16 = wh16_ref[...]
        wu = wu_ref[...]
        vh = []
        vu = []
        for c in range(3):
            h = (jnp.dot(g2[:, 16 * c:16 * (c + 1)], wh,
                         preferred_element_type=jnp.float32)
                 + xdn[:, c:c + 1] * wh16)
            vh.append(h)
            vu.append(jnp.dot(h, wu, preferred_element_type=jnp.float32))
        sh = jnp.sqrt(jnp.maximum(
            vh[0] * vh[0] + vh[1] * vh[1] + vh[2] * vh[2], 1e-8))
        pre = (g1_ref[...]
               + jnp.dot(d, wfd_ref[...], preferred_element_type=jnp.float32)
               + jnp.dot(sh, wfh_ref[...], preferred_element_type=jnp.float32))
        feats = pre * jax.nn.sigmoid(pre)
        gate = jax.nn.sigmoid(
            jnp.dot(feats, wg_ref[...], preferred_element_type=jnp.float32)
            + bg_ref[...])
        ms_ref[...] = feats
        for c in range(3):
            mv_ref[:, 16 * c:16 * (c + 1)] = gate * vu[c]
        mv_ref[:, 48:LW] = jnp.zeros((blk, LW - 48), jnp.float32)

    full = lambda shape: pl.BlockSpec(shape, lambda i: tuple(0 for _ in shape))
    return pl.pallas_call(
        body,
        grid=(e // blk,),
        in_specs=[
            pl.BlockSpec((blk, LW), lambda i: (i, 0)),
            pl.BlockSpec((blk, LW), lambda i: (i, 0)),
            pl.BlockSpec((blk, LW), lambda i: (i, 0)),
            full((DV, DV + 1)),
            full((1, DV + 1)),
            full((DV + 1, DV)),
            full((RD, DS)),
            full((DV + 1, DS)),
            full((DS, DV)),
            full((1, DV)),
            full((1, RD)),
        ],
        out_specs=(pl.BlockSpec((blk, LW), lambda i: (i, 0)),
                   pl.BlockSpec((blk, LW), lambda i: (i, 0))),
        out_shape=(jax.ShapeDtypeStruct((e, LW), jnp.float32),
                   jax.ShapeDtypeStruct((e, LW), jnp.float32)),
    )(G1, G2, Gd, Wh, Wh16, Wu, Wf_d, Wf_h, Wg, bg, dmu)


def _scatter_add(Ms, Mv, dst, zeros_blk, n_nodes):
    KS = 64
    e = Ms.shape[0]
    nchunk = e // KS
    nloop_e = (nchunk + 15) // 16
    nzfull = n_nodes // K
    ztail = n_nodes - nzfull * K
    nloop_z = (nzfull + 1 + 15) // 16
    mesh = plsc.VectorSubcoreMesh(core_axis_name="c", subcore_axis_name="s")

    @functools.partial(
        pl.kernel,
        out_type=(jax.ShapeDtypeStruct((n_nodes, LW), jnp.float32),
                  jax.ShapeDtypeStruct((n_nodes, LW), jnp.float32)),
        mesh=mesh,
        scratch_types=[
            pltpu.VMEM_SHARED((n_nodes, LW), jnp.float32),
            pltpu.VMEM((2, KS), jnp.int32),
            pltpu.VMEM((2, KS, LW), jnp.float32),
            pltpu.SemaphoreType.DMA,
            pltpu.SemaphoreType.DMA,
            pltpu.SemaphoreType.DMA,
            pltpu.SemaphoreType.DMA,
        ],
    )
    def sk(ms_hbm, mv_hbm, dst_hbm, z_hbm, outs_hbm, outv_hbm, acc, idx, rows,
           sem_i0, sem_i1, sem_m0, sem_m1):
        cid = lax.axis_index("c")
        sid = lax.axis_index("s")

        def zstep(i, _):
            j = sid + 16 * i

            @pl.when(j < nzfull)
            def _():
                pltpu.sync_copy(z_hbm, acc.at[pl.ds(j * K, K)])

            @pl.when(j == nzfull)
            def _():
                pltpu.sync_copy(z_hbm.at[pl.ds(0, ztail)],
                                acc.at[pl.ds(nzfull * K, ztail)])

            return 0

        lax.fori_loop(0, nloop_z, zstep, 0)
        plsc.subcore_barrier()

        sems = ((sem_i0, sem_m0), (sem_i1, sem_m1))

        def do_edges(m_hbm):
            def issue(j, b):
                t = sid + 16 * j

                @pl.when(t < nchunk)
                def _():
                    base = t * KS
                    pltpu.async_copy(dst_hbm.at[pl.ds(base, KS)], idx.at[b],
                                     sems[b][0])
                    pltpu.async_copy(m_hbm.at[pl.ds(base, KS)], rows.at[b],
                                     sems[b][1])

            def drain_and_scatter(j, b):
                t = sid + 16 * j

                @pl.when(t < nchunk)
                def _():
                    base = t * KS
                    pltpu.make_async_copy(dst_hbm.at[pl.ds(base, KS)],
                                          idx.at[b], sems[b][0]).wait()
                    pltpu.make_async_copy(m_hbm.at[pl.ds(base, KS)],
                                          rows.at[b], sems[b][1]).wait()
                    pltpu.sync_copy(rows.at[b], acc.at[idx.at[b]], add=True)

            issue(0, 0)
            issue(1, 1)

            def estep(i2, _):
                j0 = 2 * i2
                drain_and_scatter(j0, 0)
                issue(j0 + 2, 0)
                drain_and_scatter(j0 + 1, 1)
                issue(j0 + 3, 1)
                return 0

            lax.fori_loop(0, (nloop_e + 1) // 2, estep, 0)

        @pl.when(cid == 0)
        def _():
            do_edges(ms_hbm)

        @pl.when(cid == 1)
        def _():
            do_edges(mv_hbm)

        plsc.subcore_barrier()

        def ostep(i, _):
            j = sid + 16 * i

            @pl.when(j < nzfull)
            def _():
                @pl.when(cid == 0)
                def _():
                    pltpu.sync_copy(acc.at[pl.ds(j * K, K)],
                                    outs_hbm.at[pl.ds(j * K, K)])

                @pl.when(cid == 1)
                def _():
                    pltpu.sync_copy(acc.at[pl.ds(j * K, K)],
                                    outv_hbm.at[pl.ds(j * K, K)])

            @pl.when(j == nzfull)
            def _():
                @pl.when(cid == 0)
                def _():
                    pltpu.sync_copy(acc.at[pl.ds(nzfull * K, ztail)],
                                    outs_hbm.at[pl.ds(nzfull * K, ztail)])

                @pl.when(cid == 1)
                def _():
                    pltpu.sync_copy(acc.at[pl.ds(nzfull * K, ztail)],
                                    outv_hbm.at[pl.ds(nzfull * K, ztail)])

            return 0

        lax.fori_loop(0, nloop_z, ostep, 0)

    return sk(Ms, Mv, dst, zeros_blk)


def _node_update(scalar_feat, vt, sparts, vparts, uWh, uWu, uWf_s, uWf_h, ubf,
                 uWg, ubg, ln1_g, ln1_b, ln2_g, ln2_b):
    n = scalar_feat.shape[0]
    blk = 1000
    ns = len(sparts)
    nv = len(vparts)

    def body(sf_ref, vt_ref, *rest):
        ps_refs = rest[:ns]
        pv_refs = rest[ns:ns + nv]
        (wh_ref, wu_ref, wfs_ref, wfh_ref, bf_ref, wg_ref, bg_ref,
         g1_ref, b1_ref, g2_ref, b2_ref, so_ref, vo_ref) = rest[ns + nv:]
        sf = sf_ref[...]
        for p in ps_refs:
            sf = sf + p[...]
        vfc = []
        for c in range(3):
            v = vt_ref[:, 16 * c:16 * (c + 1)]
            for p in pv_refs:
                v = v + p[:, 16 * c:16 * (c + 1)]
            vfc.append(v)

        def ln(sf, vfc, g, b):
            mu = jnp.mean(sf, axis=1, keepdims=True)
            xc = sf - mu
            var = jnp.mean(xc * xc, axis=1, keepdims=True)
            nf = xc / jnp.sqrt(var + 1e-5) * g + b
            vn2 = vfc[0] * vfc[0] + vfc[1] * vfc[1] + vfc[2] * vfc[2]
            vnorm = jnp.sqrt(jnp.mean(vn2, axis=1, keepdims=True) + 1e-5) + 1e-5
            return nf, [v / vnorm for v in vfc]

        sf, vfc = ln(sf, vfc, g1_ref[...], b1_ref[...])
        vh = [jnp.dot(v, wh_ref[...], preferred_element_type=jnp.float32)
              for v in vfc]
        vu = [jnp.dot(h, wu_ref[...], preferred_element_type=jnp.float32)
              for h in vh]
        sh = jnp.sqrt(jnp.maximum(
            vh[0] * vh[0] + vh[1] * vh[1] + vh[2] * vh[2], 1e-8))
        pre = (jnp.dot(sf, wfs_ref[...], preferred_element_type=jnp.float32)
               + jnp.dot(sh, wfh_ref[...], preferred_element_type=jnp.float32)
               + bf_ref[...])
        feats = pre * jax.nn.sigmoid(pre)
        gate = jax.nn.sigmoid(
            jnp.dot(feats, wg_ref[...], preferred_element_type=jnp.float32)
            + bg_ref[...])
        sf2 = sf + feats
        vfc2 = [vfc[c] + gate * vu[c] for c in range(3)]
        so, voc = ln(sf2, vfc2, g2_ref[...], b2_ref[...])
        so_ref[...] = so
        for c in range(3):
            vo_ref[:, 16 * c:16 * (c + 1)] = voc[c]

    full = lambda shape: pl.BlockSpec(shape, lambda i: tuple(0 for _ in shape))
    eblk = lambda w: pl.BlockSpec((blk, w), lambda i: (i, 0))
    return pl.pallas_call(
        body,
        grid=(n // blk,),
        in_specs=[eblk(DS), eblk(3 * DV)]
        + [eblk(LW) for _ in sparts] + [eblk(LW) for _ in vparts] + [
            full((DV, DV)),
            full((DV, DV)),
            full((DS, DS)),
            full((DV, DS)),
            full((1, DS)),
            full((DS, DV)),
            full((1, DV)),
            full((1, DS)),
            full((1, DS)),
            full((1, DS)),
            full((1, DS)),
        ],
        out_specs=(eblk(DS), eblk(3 * DV)),
        out_shape=(jax.ShapeDtypeStruct((n, DS), jnp.float32),
                   jax.ShapeDtypeStruct((n, 3 * DV), jnp.float32)),
    )(scalar_feat, vt, *sparts, *vparts, uWh, uWu, uWf_s, uWf_h, ubf, uWg,
      ubg, ln1_g, ln1_b, ln2_g, ln2_b)


def kernel(scalar_feat, coord_feat, vec_feat, edge_index, msg_Wh, msg_Wu,
           msg_Wf, msg_bf, msg_Wg, msg_bg, upd_Wh, upd_Wu, upd_Wf, upd_bf,
           upd_Wg, upd_bg, ln1_g, ln1_b, ln2_g, ln2_b):
    n = scalar_feat.shape[0]
    src = edge_index[0]
    dst = edge_index[1]

    T1 = _node_precompute(scalar_feat, msg_Wf[:DS], msg_bf.reshape(1, DS))
    vt = vec_feat.transpose(0, 2, 1).reshape(n, 3 * DV)
    T2 = jnp.concatenate(
        [vt, coord_feat, jnp.zeros((n, LW - 3 * DV - 3), jnp.float32)], axis=1)
    C = jnp.concatenate(
        [coord_feat, jnp.zeros((n, LW - 3), jnp.float32)], axis=1)

    dmu = jnp.linspace(0.0, RBF_DMAX, RD).reshape(1, RD)
    zeros_blk = jnp.zeros((K, LW), jnp.float32)
    e = src.shape[0]
    half = e // 2
    sparts = []
    vparts = []
    for lo in (0, half):
        s_c = lax.dynamic_slice_in_dim(src, lo, half)
        d_c = lax.dynamic_slice_in_dim(dst, lo, half)
        G1, G2, Gd = _edge_gather(T1, T2, C, s_c, d_c)
        Ms, Mv = _edge_compute(G1, G2, Gd, msg_Wh[:DV], msg_Wh[DV:], msg_Wu,
                               msg_Wf[DS:DS + RD], msg_Wf[DS + RD:], msg_Wg,
                               msg_bg.reshape(1, DV), dmu)
        ps, pv = _scatter_add(Ms, Mv, d_c, zeros_blk, n)
        sparts.append(ps)
        vparts.append(pv)

    so, vo = _node_update(
        scalar_feat, vt, sparts, vparts, upd_Wh, upd_Wu, upd_Wf[:DS],
        upd_Wf[DS:], upd_bf.reshape(1, DS), upd_Wg, upd_bg.reshape(1, DV),
        ln1_g.reshape(1, DS), ln1_b.reshape(1, DS),
        ln2_g.reshape(1, DS), ln2_b.reshape(1, DS))
    vout = vo.reshape(n, 3, DV).transpose(0, 2, 1)
    return so, vout
